# Initial kernel scaffold; baseline (speedup 1.0000x reference)
import jax
import jax.numpy as jnp
from jax.experimental import pallas as pl


def kernel(token_ids, weight):
    raise NotImplementedError("write your pallas kernel here")


# speedup vs baseline: 1.0937x; 1.0937x over previous
import functools

import jax
import jax.numpy as jnp
from jax import lax
from jax.experimental import pallas as pl
from jax.experimental.pallas import tpu as pltpu
from jax.experimental.pallas import tpu_sc as plsc

NUM_ROWS_TABLE = 1000000
D = 32
NC = 2
NS = 16
NW = NC * NS
CHUNK = 128
GROUP = 8
B = 16384 * 50
IDX_ROWS = B // CHUNK
ROWS_PER_W = IDX_ROWS // NW
NGROUPS = ROWS_PER_W // GROUP


def _body(idx_hbm, table_hbm, out_hbm, idx_v, rows_v, sem):
    wid = lax.axis_index("s") * NC + lax.axis_index("c")
    row0 = wid * ROWS_PER_W

    def group(g, carry):
        r = row0 + g * GROUP
        pltpu.sync_copy(idx_hbm.at[pl.ds(r, GROUP)], idx_v)
        copies = []
        for j in range(GROUP):
            cp = pltpu.async_copy(
                table_hbm.at[idx_v.at[j]],
                rows_v.at[pl.ds(j * CHUNK, CHUNK)],
                sem,
            )
            copies.append(cp)
        for cp in copies:
            cp.wait()
        pltpu.sync_copy(rows_v, out_hbm.at[pl.ds(r * CHUNK, GROUP * CHUNK)])
        return carry

    lax.fori_loop(0, NGROUPS, group, 0)


@functools.partial(jax.jit, static_argnums=())
def _lookup(idx2d, weight):
    mesh = plsc.VectorSubcoreMesh(
        core_axis_name="c", subcore_axis_name="s", num_cores=NC, num_subcores=NS
    )
    f = pl.kernel(
        _body,
        out_type=jax.ShapeDtypeStruct((B, D), jnp.float32),
        mesh=mesh,
        scratch_types=[
            pltpu.VMEM((GROUP, CHUNK), jnp.int32),
            pltpu.VMEM((GROUP * CHUNK, D), jnp.float32),
            pltpu.SemaphoreType.DMA,
        ],
        compiler_params=pltpu.CompilerParams(use_tc_tiling_on_sc=False),
    )
    return f(idx2d, weight)


def kernel(token_ids, weight):
    s0, s1 = token_ids.shape
    idx2d = token_ids.astype(jnp.int32).reshape(IDX_ROWS, CHUNK)
    out = _lookup(idx2d, weight)
    return out.reshape(s0, s1, D)


# speedup vs baseline: 1.1097x; 1.0146x over previous
import functools

import jax
import jax.numpy as jnp
from jax import lax
from jax.experimental import pallas as pl
from jax.experimental.pallas import tpu as pltpu
from jax.experimental.pallas import tpu_sc as plsc

D = 32
NC = 2
NS = 16
NW = NC * NS
CHUNK = 128
GROUP = 10
B = 16384 * 50
IDX_ROWS = B // CHUNK
ROWS_PER_W = IDX_ROWS // NW
NGROUPS = ROWS_PER_W // GROUP
NPAIRS = NGROUPS // 2


def _body(idx_hbm, table_hbm, out_hbm, idx_v0, idx_v1, rows_v0, rows_v1,
          sem_i0, sem_i1, sem_g, sem_o0, sem_o1):
    wid = lax.axis_index("s") * NC + lax.axis_index("c")
    row0 = wid * ROWS_PER_W

    def idx_slice(g):
        return idx_hbm.at[pl.ds(row0 + g * GROUP, GROUP)]

    def out_slice(g):
        return out_hbm.at[pl.ds((row0 + g * GROUP) * CHUNK, GROUP * CHUNK)]

    def gather_group(idx_v, rows_v):
        copies = [
            pltpu.async_copy(
                table_hbm.at[idx_v.at[j]],
                rows_v.at[pl.ds(j * CHUNK, CHUNK)],
                sem_g,
            )
            for j in range(GROUP)
        ]
        for cp in copies:
            cp.wait()

    def drain_out(rows_v, sem):
        pltpu.make_async_copy(out_slice(0), rows_v, sem).wait()

    pltpu.async_copy(idx_slice(0), idx_v0, sem_i0)

    def pair(p, carry):
        g0 = 2 * p
        g1 = g0 + 1

        pltpu.make_async_copy(idx_slice(g0), idx_v0, sem_i0).wait()

        @pl.when(p > 0)
        def _():
            drain_out(rows_v0, sem_o0)

        pltpu.async_copy(idx_slice(g1), idx_v1, sem_i1)
        gather_group(idx_v0, rows_v0)
        pltpu.async_copy(rows_v0, out_slice(g0), sem_o0)

        pltpu.make_async_copy(idx_slice(g1), idx_v1, sem_i1).wait()

        @pl.when(p > 0)
        def _():
            drain_out(rows_v1, sem_o1)

        @pl.when(p + 1 < NPAIRS)
        def _():
            pltpu.async_copy(idx_slice(g0 + 2), idx_v0, sem_i0)

        gather_group(idx_v1, rows_v1)
        pltpu.async_copy(rows_v1, out_slice(g1), sem_o1)
        return carry

    lax.fori_loop(0, NPAIRS, pair, 0)

    drain_out(rows_v0, sem_o0)
    drain_out(rows_v1, sem_o1)


@jax.jit
def _lookup(idx2d, weight):
    mesh = plsc.VectorSubcoreMesh(
        core_axis_name="c", subcore_axis_name="s", num_cores=NC, num_subcores=NS
    )
    f = pl.kernel(
        _body,
        out_type=jax.ShapeDtypeStruct((B, D), jnp.float32),
        mesh=mesh,
        scratch_types=[
            pltpu.VMEM((GROUP, CHUNK), jnp.int32),
            pltpu.VMEM((GROUP, CHUNK), jnp.int32),
            pltpu.VMEM((GROUP * CHUNK, D), jnp.float32),
            pltpu.VMEM((GROUP * CHUNK, D), jnp.float32),
            pltpu.SemaphoreType.DMA,
            pltpu.SemaphoreType.DMA,
            pltpu.SemaphoreType.DMA,
            pltpu.SemaphoreType.DMA,
            pltpu.SemaphoreType.DMA,
        ],
        compiler_params=pltpu.CompilerParams(use_tc_tiling_on_sc=False),
    )
    return f(idx2d, weight)


def kernel(token_ids, weight):
    s0, s1 = token_ids.shape
    idx2d = token_ids.astype(jnp.int32).reshape(IDX_ROWS, CHUNK)
    out = _lookup(idx2d, weight)
    return out.reshape(s0, s1, D)
